# Initial kernel scaffold; baseline (speedup 1.0000x reference)
import jax
import jax.numpy as jnp
from jax.experimental import pallas as pl


def kernel(node_prediction, edge_prediction, edge_index, track_labels, track_vtx, track_node_idx, lep_labels, lep_vtx, lep_node_idx, cell_labels, cell_vtx, cell_node_idx, node_graph_ids, edge_graph_ids):
    raise NotImplementedError("write your pallas kernel here")


# speedup vs baseline: 206.6191x; 206.6191x over previous
import jax
import jax.numpy as jnp
from jax import lax
from jax.experimental import pallas as pl
from jax.experimental.pallas import tpu as pltpu
from jax.experimental.pallas import tpu_sc as plsc

N_NODES = 100000
N_GRAPHS = 16
NC = 2
NS = 16
NW = NC * NS
N_PAD = 100352
TAB_SLICE = N_PAD // NS

N_ITEMS = 310000
ITEM_PAD = 327680
IROWS = ITEM_PAD // 128
IROWS_PW = IROWS // NW
ILOADS = IROWS_PW // 16

E_TOTAL = 6400000
E_PW = E_TOTAL // NW
E_CHUNK = 2000
E_NCHUNK = E_PW // E_CHUNK
E_STEPS = E_CHUNK // 16

TCB = 2048

_LOG1P_C = (
    3.910905377324525e-08, 0.999993622303009, -0.49982550740242004,
    0.33144664764404297, -0.2394333779811859, 0.1649981290102005,
    -0.09229041635990143, 0.03426460176706314, -0.006006604991853237,
)


def _scatter_body(idx_hbm, lab_hbm, vtx_hbm, lab_out, vtx_out,
                  idx_b, lab_b, vtx_b, stage, lab_t, vtx_t):
    cid = lax.axis_index("c")
    sid = lax.axis_index("s")
    wid = sid * NC + cid
    def z(i, carry):
        stage[pl.ds(i * 16, 16)] = jnp.zeros((16,), jnp.int32)
        return carry
    lax.fori_loop(0, TAB_SLICE // 16, z, 0)
    off = sid * TAB_SLICE
    pltpu.sync_copy(stage, lab_t.at[pl.ds(off, TAB_SLICE)])
    pltpu.sync_copy(stage, vtx_t.at[pl.ds(off, TAB_SLICE)])
    plsc.subcore_barrier()
    for l in range(ILOADS):
        r0 = wid * IROWS_PW + l * 16
        pltpu.sync_copy(idx_hbm.at[pl.ds(r0, 16)], idx_b)
        pltpu.sync_copy(lab_hbm.at[pl.ds(r0, 16)], lab_b)
        pltpu.sync_copy(vtx_hbm.at[pl.ds(r0, 16)], vtx_b)
        for j in range(16):
            pltpu.sync_copy(lab_b.at[j], lab_t.at[idx_b.at[j]], add=True)
            pltpu.sync_copy(vtx_b.at[j], vtx_t.at[idx_b.at[j]], add=True)
    plsc.subcore_barrier()
    pltpu.sync_copy(lab_t.at[pl.ds(off, TAB_SLICE)], stage)
    pltpu.sync_copy(stage, lab_out.at[cid, pl.ds(off, TAB_SLICE)])
    pltpu.sync_copy(vtx_t.at[pl.ds(off, TAB_SLICE)], stage)
    pltpu.sync_copy(stage, vtx_out.at[cid, pl.ds(off, TAB_SLICE)])


_CALL_CACHE = {}


def _get_scatter_call():
    if "scatter" not in _CALL_CACHE:
        _CALL_CACHE["scatter"] = pl.kernel(
            _scatter_body,
            out_type=[jax.ShapeDtypeStruct((NC, N_PAD), jnp.int32),
                      jax.ShapeDtypeStruct((NC, N_PAD), jnp.int32)],
            mesh=plsc.VectorSubcoreMesh(core_axis_name="c",
                                        subcore_axis_name="s",
                                        num_cores=NC, num_subcores=NS),
            compiler_params=pltpu.CompilerParams(needs_layout_passes=False),
            scratch_types=[
                pltpu.VMEM((16, 128), jnp.int32),
                pltpu.VMEM((16, 128), jnp.int32),
                pltpu.VMEM((16, 128), jnp.int32),
                pltpu.VMEM((TAB_SLICE,), jnp.int32),
                pltpu.VMEM_SHARED((N_PAD,), jnp.int32),
                pltpu.VMEM_SHARED((N_PAD,), jnp.int32),
            ],
        )
    return _CALL_CACHE["scatter"]


def _node_body(pred_ref, lab_ref, vtx_ref, loss_ref, packed_ref):
    i = pl.program_id(0)
    x = pred_ref[...]
    m = jnp.max(x, axis=0, keepdims=True)
    lse = m + jnp.log(jnp.sum(jnp.exp(x - m), axis=0, keepdims=True))
    lab = lab_ref[0:1, :] + lab_ref[1:2, :]
    vtxv = vtx_ref[0:1, :] + vtx_ref[1:2, :]
    lblc = jnp.clip(lab, 0, 4)
    rows = lax.broadcasted_iota(jnp.int32, x.shape, 0)
    sel = jnp.sum(jnp.where(rows == lblc, x, 0.0), axis=0, keepdims=True)
    nll = lse - sel
    col = i * TCB + lax.broadcasted_iota(jnp.int32, (1, TCB), 1)
    ok = (col < N_NODES) & (lab != -1)
    nz = jnp.where(ok, nll, 0.0).reshape(16, 128)
    part = nz[:8, :] + nz[8:, :]

    @pl.when(i == 0)
    def _():
        loss_ref[...] = jnp.zeros((8, 128), jnp.float32)

    loss_ref[...] += part
    cond = ((lab == 2) | (lab == 3)).astype(jnp.int32)
    packed_ref[...] = vtxv * 2 + cond


_node_call = pl.pallas_call(
    _node_body,
    grid=(N_PAD // TCB,),
    in_specs=[pl.BlockSpec((5, TCB), lambda i: (0, i)),
              pl.BlockSpec((2, TCB), lambda i: (0, i)),
              pl.BlockSpec((2, TCB), lambda i: (0, i))],
    out_specs=[pl.BlockSpec((8, 128), lambda i: (0, 0)),
               pl.BlockSpec((1, TCB), lambda i: (0, i))],
    out_shape=[jax.ShapeDtypeStruct((8, 128), jnp.float32),
               jax.ShapeDtypeStruct((1, N_PAD), jnp.int32)],
)


def _edge_body(packed_hbm, x_hbm, eidx_hbm, gid_hbm, out_hbm,
               table, src_b, dst_b, x_b, gid_b, acc, outbuf):
    cid = lax.axis_index("c")
    sid = lax.axis_index("s")
    wid = sid * NC + cid
    pltpu.sync_copy(packed_hbm, table)
    for q in range(5):
        for i in range(16):
            acc[q, i, :] = jnp.zeros((16,), jnp.float32)
    lane = lax.iota(jnp.int32, 16)
    ones16 = jnp.full((16,), 1.0, jnp.float32)
    base_w = wid * E_PW

    def chunk(c, carry):
        base = base_w + c * E_CHUNK
        pltpu.sync_copy(eidx_hbm.at[pl.ds(base, E_CHUNK)], src_b)
        pltpu.sync_copy(eidx_hbm.at[pl.ds(E_TOTAL + base, E_CHUNK)], dst_b)
        pltpu.sync_copy(x_hbm.at[pl.ds(base, E_CHUNK)], x_b)
        pltpu.sync_copy(gid_hbm.at[pl.ds(base, E_CHUNK)], gid_b)

        def step(i, c2):
            sl = pl.ds(i * 16, 16)
            s = src_b[sl]
            d = dst_b[sl]
            x = x_b[sl]
            g = gid_b[sl]
            ps = plsc.load_gather(table, [s])
            pd = plsc.load_gather(table, [d])
            y = jnp.where((ps >> 1) == (pd >> 1), 1.0, 0.0).astype(jnp.float32)
            w = jnp.where((ps & pd & 1) == 1, 2.0, 1.0).astype(jnp.float32)
            t = jnp.exp(-jnp.abs(x))
            p = jnp.full((16,), _LOG1P_C[8], jnp.float32)
            for k in range(7, -1, -1):
                p = p * t + _LOG1P_C[k]
            r = 1.0 / (1.0 + t)
            yhat = jnp.where(x >= 0, r, 1.0 - r)
            bce = (jnp.maximum(x, 0.0) - x * y + p) * w
            yw = y * w
            yhw = yhat * w
            tp = yhat * yw
            plsc.addupdate_scatter(acc.at[0], [lane, g], bce)
            plsc.addupdate_scatter(acc.at[1], [lane, g], ones16)
            plsc.addupdate_scatter(acc.at[2], [lane, g], tp)
            plsc.addupdate_scatter(acc.at[3], [lane, g], yw)
            plsc.addupdate_scatter(acc.at[4], [lane, g], yhw)
            return c2
        lax.fori_loop(0, E_STEPS, step, 0)
        return carry
    lax.fori_loop(0, E_NCHUNK, chunk, 0)

    for q in range(5):
        v = acc[q, 0, :]
        for i in range(1, 16):
            v = v + acc[q, i, :]
        outbuf[q, :] = v
    pltpu.sync_copy(outbuf, out_hbm.at[wid])


def _get_edge_call():
    if "edge" not in _CALL_CACHE:
        _CALL_CACHE["edge"] = pl.kernel(
            _edge_body,
            out_type=jax.ShapeDtypeStruct((NW, 5, 16), jnp.float32),
            mesh=plsc.VectorSubcoreMesh(core_axis_name="c",
                                        subcore_axis_name="s",
                                        num_cores=NC, num_subcores=NS),
            compiler_params=pltpu.CompilerParams(needs_layout_passes=False),
            scratch_types=[
                pltpu.VMEM((N_PAD,), jnp.int32),
                pltpu.VMEM((E_CHUNK,), jnp.int32),
                pltpu.VMEM((E_CHUNK,), jnp.int32),
                pltpu.VMEM((E_CHUNK,), jnp.float32),
                pltpu.VMEM((E_CHUNK,), jnp.int32),
                pltpu.VMEM((5, 16, 16), jnp.float32),
                pltpu.VMEM((5, 16), jnp.float32),
            ],
        )
    return _CALL_CACHE["edge"]


def kernel(node_prediction, edge_prediction, edge_index, track_labels,
           track_vtx, track_node_idx, lep_labels, lep_vtx, lep_node_idx,
           cell_labels, cell_vtx, cell_node_idx, node_graph_ids,
           edge_graph_ids):
    idx = jnp.concatenate([track_node_idx, lep_node_idx, cell_node_idx])
    lab = jnp.concatenate([track_labels, lep_labels, cell_labels])
    vtx = jnp.concatenate([track_vtx, lep_vtx, cell_vtx])
    padn = ITEM_PAD - N_ITEMS
    idx = jnp.pad(idx.astype(jnp.int32), (0, padn)).reshape(IROWS, 128)
    lab = jnp.pad(lab.astype(jnp.int32), (0, padn)).reshape(IROWS, 128)
    vtx = jnp.pad(vtx.astype(jnp.int32), (0, padn)).reshape(IROWS, 128)

    lab_part, vtx_part = _get_scatter_call()(idx, lab, vtx)

    predT = jnp.pad(node_prediction, ((0, N_PAD - N_NODES), (0, 0))).T
    loss_parts, packed = _node_call(predT, lab_part, vtx_part)

    acc = _get_edge_call()(packed.reshape(N_PAD),
                     edge_prediction,
                     edge_index.astype(jnp.int32).reshape(2 * E_TOTAL),
                     edge_graph_ids.astype(jnp.int32))

    parts = acc.sum(axis=0)
    bce_sum = parts[0]
    cnt = parts[1]
    tp = parts[2]
    yw = parts[3]
    yhw = parts[4]
    fn = yw - tp
    fp = yhw - tp
    node_loss = loss_parts.sum() / N_GRAPHS
    edge_bce = (bce_sum / jnp.maximum(cnt, 1.0)).mean()
    edge_f1 = -(2.0 * tp / (2.0 * tp + fp + fn + 1e-10)).mean()
    loss = node_loss + edge_bce + edge_f1
    return (loss, node_loss, edge_bce, edge_f1)


# speedup vs baseline: 366.6121x; 1.7743x over previous
import jax
import jax.numpy as jnp
from jax import lax
from jax.experimental import pallas as pl
from jax.experimental.pallas import tpu as pltpu
from jax.experimental.pallas import tpu_sc as plsc

N_NODES = 100000
N_GRAPHS = 16
NC = 2
NS = 16
NW = NC * NS
N_PAD = 100352
TAB_SLICE = N_PAD // NS

N_ITEMS = 310000
ITEM_PAD = 327680
IROWS = ITEM_PAD // 128
IROWS_PW = IROWS // NW
ILOADS = IROWS_PW // 16

E_TOTAL = 6400000
E_PW = E_TOTAL // NW
E_CHUNK = 2000
E_NCHUNK = E_PW // E_CHUNK
E_STEPS = E_CHUNK // 16

TCB = 2048

_LOG1P_C = (
    3.910905377324525e-08, 0.999993622303009, -0.49982550740242004,
    0.33144664764404297, -0.2394333779811859, 0.1649981290102005,
    -0.09229041635990143, 0.03426460176706314, -0.006006604991853237,
)


def _scatter_body(idx_hbm, lab_hbm, vtx_hbm, lab_out, vtx_out,
                  idx_b, lab_b, vtx_b, stage, lab_t, vtx_t):
    cid = lax.axis_index("c")
    sid = lax.axis_index("s")
    wid = sid * NC + cid
    def z(i, carry):
        stage[pl.ds(i * 16, 16)] = jnp.zeros((16,), jnp.int32)
        return carry
    lax.fori_loop(0, TAB_SLICE // 16, z, 0)
    off = sid * TAB_SLICE
    pltpu.sync_copy(stage, lab_t.at[pl.ds(off, TAB_SLICE)])
    pltpu.sync_copy(stage, vtx_t.at[pl.ds(off, TAB_SLICE)])
    plsc.subcore_barrier()
    for l in range(ILOADS):
        r0 = wid * IROWS_PW + l * 16
        pltpu.sync_copy(idx_hbm.at[pl.ds(r0, 16)], idx_b)
        pltpu.sync_copy(lab_hbm.at[pl.ds(r0, 16)], lab_b)
        pltpu.sync_copy(vtx_hbm.at[pl.ds(r0, 16)], vtx_b)
        for j in range(16):
            pltpu.sync_copy(lab_b.at[j], lab_t.at[idx_b.at[j]], add=True)
            pltpu.sync_copy(vtx_b.at[j], vtx_t.at[idx_b.at[j]], add=True)
    plsc.subcore_barrier()
    pltpu.sync_copy(lab_t.at[pl.ds(off, TAB_SLICE)], stage)
    pltpu.sync_copy(stage, lab_out.at[cid, pl.ds(off, TAB_SLICE)])
    pltpu.sync_copy(vtx_t.at[pl.ds(off, TAB_SLICE)], stage)
    pltpu.sync_copy(stage, vtx_out.at[cid, pl.ds(off, TAB_SLICE)])


_CALL_CACHE = {}


def _get_scatter_call():
    if "scatter" not in _CALL_CACHE:
        _CALL_CACHE["scatter"] = pl.kernel(
            _scatter_body,
            out_type=[jax.ShapeDtypeStruct((NC, N_PAD), jnp.int32),
                      jax.ShapeDtypeStruct((NC, N_PAD), jnp.int32)],
            mesh=plsc.VectorSubcoreMesh(core_axis_name="c",
                                        subcore_axis_name="s",
                                        num_cores=NC, num_subcores=NS),
            compiler_params=pltpu.CompilerParams(needs_layout_passes=False),
            scratch_types=[
                pltpu.VMEM((16, 128), jnp.int32),
                pltpu.VMEM((16, 128), jnp.int32),
                pltpu.VMEM((16, 128), jnp.int32),
                pltpu.VMEM((TAB_SLICE,), jnp.int32),
                pltpu.VMEM_SHARED((N_PAD,), jnp.int32),
                pltpu.VMEM_SHARED((N_PAD,), jnp.int32),
            ],
        )
    return _CALL_CACHE["scatter"]


def _node_body(pred_ref, lab_ref, vtx_ref, loss_ref, packed_ref):
    i = pl.program_id(0)
    x = pred_ref[...]
    m = jnp.max(x, axis=0, keepdims=True)
    lse = m + jnp.log(jnp.sum(jnp.exp(x - m), axis=0, keepdims=True))
    lab = lab_ref[0:1, :] + lab_ref[1:2, :]
    vtxv = vtx_ref[0:1, :] + vtx_ref[1:2, :]
    lblc = jnp.clip(lab, 0, 4)
    rows = lax.broadcasted_iota(jnp.int32, x.shape, 0)
    sel = jnp.sum(jnp.where(rows == lblc, x, 0.0), axis=0, keepdims=True)
    nll = lse - sel
    col = i * TCB + lax.broadcasted_iota(jnp.int32, (1, TCB), 1)
    ok = (col < N_NODES) & (lab != -1)
    nz = jnp.where(ok, nll, 0.0).reshape(16, 128)
    part = nz[:8, :] + nz[8:, :]

    @pl.when(i == 0)
    def _():
        loss_ref[...] = jnp.zeros((8, 128), jnp.float32)

    loss_ref[...] += part
    cond = ((lab == 2) | (lab == 3)).astype(jnp.int32)
    packed_ref[...] = vtxv * 2 + cond


_node_call = pl.pallas_call(
    _node_body,
    grid=(N_PAD // TCB,),
    in_specs=[pl.BlockSpec((5, TCB), lambda i: (0, i)),
              pl.BlockSpec((2, TCB), lambda i: (0, i)),
              pl.BlockSpec((2, TCB), lambda i: (0, i))],
    out_specs=[pl.BlockSpec((8, 128), lambda i: (0, 0)),
               pl.BlockSpec((1, TCB), lambda i: (0, i))],
    out_shape=[jax.ShapeDtypeStruct((8, 128), jnp.float32),
               jax.ShapeDtypeStruct((1, N_PAD), jnp.int32)],
)


def _edge_body(packed_hbm, x_hbm, eidx_hbm, gid_hbm, out_hbm,
               table, src_b0, dst_b0, x_b0, gid_b0,
               src_b1, dst_b1, x_b1, gid_b1, acc, outbuf, sems):
    cid = lax.axis_index("c")
    sid = lax.axis_index("s")
    wid = sid * NC + cid
    bufs = ((src_b0, dst_b0, x_b0, gid_b0), (src_b1, dst_b1, x_b1, gid_b1))
    pltpu.sync_copy(packed_hbm, table)
    for q in range(5):
        for i in range(16):
            acc[q, i, :] = jnp.zeros((16,), jnp.float32)
    lane = lax.iota(jnp.int32, 16)
    ones16 = jnp.full((16,), 1.0, jnp.float32)
    base_w = wid * E_PW

    def issue(c, bp):
        sb, db, xb, gb = bufs[bp]
        base = base_w + c * E_CHUNK
        pltpu.async_copy(eidx_hbm.at[pl.ds(base, E_CHUNK)], sb, sems.at[bp])
        pltpu.async_copy(eidx_hbm.at[pl.ds(E_TOTAL + base, E_CHUNK)],
                         db, sems.at[bp])
        pltpu.async_copy(x_hbm.at[pl.ds(base, E_CHUNK)], xb, sems.at[bp])
        pltpu.async_copy(gid_hbm.at[pl.ds(base, E_CHUNK)], gb, sems.at[bp])

    def wait(bp):
        sb, db, xb, gb = bufs[bp]
        pltpu.make_async_copy(eidx_hbm.at[pl.ds(0, E_CHUNK)],
                              sb, sems.at[bp]).wait()
        pltpu.make_async_copy(eidx_hbm.at[pl.ds(0, E_CHUNK)],
                              db, sems.at[bp]).wait()
        pltpu.make_async_copy(x_hbm.at[pl.ds(0, E_CHUNK)],
                              xb, sems.at[bp]).wait()
        pltpu.make_async_copy(gid_hbm.at[pl.ds(0, E_CHUNK)],
                              gb, sems.at[bp]).wait()

    def process(bp):
        sb, db, xb, gb = bufs[bp]

        @plsc.parallel_loop(0, E_STEPS, unroll=5)
        def _(i):
            sl = pl.ds(i * 16, 16)
            s = sb[sl]
            d = db[sl]
            x = xb[sl]
            g = gb[sl]
            ps = plsc.load_gather(table, [s])
            pd = plsc.load_gather(table, [d])
            y = jnp.where((ps >> 1) == (pd >> 1), 1.0, 0.0).astype(jnp.float32)
            w = jnp.where((ps & pd & 1) == 1, 2.0, 1.0).astype(jnp.float32)
            t = jnp.exp(-jnp.abs(x))
            p = jnp.full((16,), _LOG1P_C[8], jnp.float32)
            for k in range(7, -1, -1):
                p = p * t + _LOG1P_C[k]
            r = 1.0 / (1.0 + t)
            yhat = jnp.where(x >= 0, r, 1.0 - r)
            bce = (jnp.maximum(x, 0.0) - x * y + p) * w
            yw = y * w
            yhw = yhat * w
            tp = yhat * yw
            plsc.addupdate_scatter(acc.at[0], [lane, g], bce)
            plsc.addupdate_scatter(acc.at[1], [lane, g], ones16)
            plsc.addupdate_scatter(acc.at[2], [lane, g], tp)
            plsc.addupdate_scatter(acc.at[3], [lane, g], yw)
            plsc.addupdate_scatter(acc.at[4], [lane, g], yhw)

    issue(0, 0)
    issue(1, 1)

    def outer(cc, carry):
        c0 = 2 * cc
        wait(0)
        process(0)
        issue(jnp.minimum(c0 + 2, E_NCHUNK - 1), 0)
        wait(1)
        process(1)
        issue(jnp.minimum(c0 + 3, E_NCHUNK - 1), 1)
        return carry
    lax.fori_loop(0, E_NCHUNK // 2, outer, 0)
    wait(0)
    wait(1)

    for q in range(5):
        v = acc[q, 0, :]
        for i in range(1, 16):
            v = v + acc[q, i, :]
        outbuf[q, :] = v
    pltpu.sync_copy(outbuf, out_hbm.at[wid])


def _get_edge_call():
    if "edge" not in _CALL_CACHE:
        _CALL_CACHE["edge"] = pl.kernel(
            _edge_body,
            out_type=jax.ShapeDtypeStruct((NW, 5, 16), jnp.float32),
            mesh=plsc.VectorSubcoreMesh(core_axis_name="c",
                                        subcore_axis_name="s",
                                        num_cores=NC, num_subcores=NS),
            compiler_params=pltpu.CompilerParams(needs_layout_passes=False),
            scratch_types=[
                pltpu.VMEM((N_PAD,), jnp.int32),
                pltpu.VMEM((E_CHUNK,), jnp.int32),
                pltpu.VMEM((E_CHUNK,), jnp.int32),
                pltpu.VMEM((E_CHUNK,), jnp.float32),
                pltpu.VMEM((E_CHUNK,), jnp.int32),
                pltpu.VMEM((E_CHUNK,), jnp.int32),
                pltpu.VMEM((E_CHUNK,), jnp.int32),
                pltpu.VMEM((E_CHUNK,), jnp.float32),
                pltpu.VMEM((E_CHUNK,), jnp.int32),
                pltpu.VMEM((5, 16, 16), jnp.float32),
                pltpu.VMEM((5, 16), jnp.float32),
                pltpu.SemaphoreType.DMA((2,)),
            ],
        )
    return _CALL_CACHE["edge"]


def kernel(node_prediction, edge_prediction, edge_index, track_labels,
           track_vtx, track_node_idx, lep_labels, lep_vtx, lep_node_idx,
           cell_labels, cell_vtx, cell_node_idx, node_graph_ids,
           edge_graph_ids):
    idx = jnp.concatenate([track_node_idx, lep_node_idx, cell_node_idx])
    lab = jnp.concatenate([track_labels, lep_labels, cell_labels])
    vtx = jnp.concatenate([track_vtx, lep_vtx, cell_vtx])
    padn = ITEM_PAD - N_ITEMS
    idx = jnp.pad(idx.astype(jnp.int32), (0, padn)).reshape(IROWS, 128)
    lab = jnp.pad(lab.astype(jnp.int32), (0, padn)).reshape(IROWS, 128)
    vtx = jnp.pad(vtx.astype(jnp.int32), (0, padn)).reshape(IROWS, 128)

    lab_part, vtx_part = _get_scatter_call()(idx, lab, vtx)

    predT = jnp.pad(node_prediction, ((0, N_PAD - N_NODES), (0, 0))).T
    loss_parts, packed = _node_call(predT, lab_part, vtx_part)

    acc = _get_edge_call()(packed.reshape(N_PAD),
                     edge_prediction,
                     edge_index.astype(jnp.int32).reshape(2 * E_TOTAL),
                     edge_graph_ids.astype(jnp.int32))

    parts = acc.sum(axis=0)
    bce_sum = parts[0]
    cnt = parts[1]
    tp = parts[2]
    yw = parts[3]
    yhw = parts[4]
    fn = yw - tp
    fp = yhw - tp
    node_loss = loss_parts.sum() / N_GRAPHS
    edge_bce = (bce_sum / jnp.maximum(cnt, 1.0)).mean()
    edge_f1 = -(2.0 * tp / (2.0 * tp + fp + fn + 1e-10)).mean()
    loss = node_loss + edge_bce + edge_f1
    return (loss, node_loss, edge_bce, edge_f1)


# speedup vs baseline: 541.8539x; 1.4780x over previous
import jax
import jax.numpy as jnp
from jax import lax
from jax.experimental import pallas as pl
from jax.experimental.pallas import tpu as pltpu
from jax.experimental.pallas import tpu_sc as plsc

N_NODES = 100000
N_GRAPHS = 16
NC = 2
NS = 16
NW = NC * NS
N_PAD = 100352
TAB_SLICE = N_PAD // NS

N_ITEMS = 310000
ITEM_PAD = 327680
IROWS = ITEM_PAD // 128
IROWS_PW = IROWS // NW
ILOADS = IROWS_PW // 16

E_TOTAL = 6400000
E_PW = E_TOTAL // NW
E_CHUNK = 2000
E_NCHUNK = E_PW // E_CHUNK
E_STEPS = E_CHUNK // 16

TCB = 2048

N_BANKS = 5
ACC_WORDS = N_BANKS * 5 * 256

_LOG1P_C = (
    3.910905377324525e-08, 0.999993622303009, -0.49982550740242004,
    0.33144664764404297, -0.2394333779811859, 0.1649981290102005,
    -0.09229041635990143, 0.03426460176706314, -0.006006604991853237,
)


def _scatter_body(idx_hbm, lab_hbm, vtx_hbm, lab_out, vtx_out,
                  idx_b, lab_b, vtx_b, stage, lab_t, vtx_t):
    cid = lax.axis_index("c")
    sid = lax.axis_index("s")
    wid = sid * NC + cid
    def z(i, carry):
        stage[pl.ds(i * 16, 16)] = jnp.zeros((16,), jnp.int32)
        return carry
    lax.fori_loop(0, TAB_SLICE // 16, z, 0)
    off = sid * TAB_SLICE
    pltpu.sync_copy(stage, lab_t.at[pl.ds(off, TAB_SLICE)])
    pltpu.sync_copy(stage, vtx_t.at[pl.ds(off, TAB_SLICE)])
    plsc.subcore_barrier()
    for l in range(ILOADS):
        r0 = wid * IROWS_PW + l * 16
        pltpu.sync_copy(idx_hbm.at[pl.ds(r0, 16)], idx_b)
        pltpu.sync_copy(lab_hbm.at[pl.ds(r0, 16)], lab_b)
        pltpu.sync_copy(vtx_hbm.at[pl.ds(r0, 16)], vtx_b)
        for j in range(16):
            pltpu.sync_copy(lab_b.at[j], lab_t.at[idx_b.at[j]], add=True)
            pltpu.sync_copy(vtx_b.at[j], vtx_t.at[idx_b.at[j]], add=True)
    plsc.subcore_barrier()
    pltpu.sync_copy(lab_t.at[pl.ds(off, TAB_SLICE)], stage)
    pltpu.sync_copy(stage, lab_out.at[cid, pl.ds(off, TAB_SLICE)])
    pltpu.sync_copy(vtx_t.at[pl.ds(off, TAB_SLICE)], stage)
    pltpu.sync_copy(stage, vtx_out.at[cid, pl.ds(off, TAB_SLICE)])


_CALL_CACHE = {}


def _get_scatter_call():
    if "scatter" not in _CALL_CACHE:
        _CALL_CACHE["scatter"] = pl.kernel(
            _scatter_body,
            out_type=[jax.ShapeDtypeStruct((NC, N_PAD), jnp.int32),
                      jax.ShapeDtypeStruct((NC, N_PAD), jnp.int32)],
            mesh=plsc.VectorSubcoreMesh(core_axis_name="c",
                                        subcore_axis_name="s",
                                        num_cores=NC, num_subcores=NS),
            compiler_params=pltpu.CompilerParams(needs_layout_passes=False),
            scratch_types=[
                pltpu.VMEM((16, 128), jnp.int32),
                pltpu.VMEM((16, 128), jnp.int32),
                pltpu.VMEM((16, 128), jnp.int32),
                pltpu.VMEM((TAB_SLICE,), jnp.int32),
                pltpu.VMEM_SHARED((N_PAD,), jnp.int32),
                pltpu.VMEM_SHARED((N_PAD,), jnp.int32),
            ],
        )
    return _CALL_CACHE["scatter"]


def _node_body(pred_ref, lab_ref, vtx_ref, loss_ref, packed_ref):
    i = pl.program_id(0)
    x = pred_ref[...]
    m = jnp.max(x, axis=0, keepdims=True)
    lse = m + jnp.log(jnp.sum(jnp.exp(x - m), axis=0, keepdims=True))
    lab = lab_ref[0:1, :] + lab_ref[1:2, :]
    vtxv = vtx_ref[0:1, :] + vtx_ref[1:2, :]
    lblc = jnp.clip(lab, 0, 4)
    rows = lax.broadcasted_iota(jnp.int32, x.shape, 0)
    sel = jnp.sum(jnp.where(rows == lblc, x, 0.0), axis=0, keepdims=True)
    nll = lse - sel
    col = i * TCB + lax.broadcasted_iota(jnp.int32, (1, TCB), 1)
    ok = (col < N_NODES) & (lab != -1)
    nz = jnp.where(ok, nll, 0.0).reshape(16, 128)
    part = nz[:8, :] + nz[8:, :]

    @pl.when(i == 0)
    def _():
        loss_ref[...] = jnp.zeros((8, 128), jnp.float32)

    loss_ref[...] += part
    cond = ((lab == 2) | (lab == 3)).astype(jnp.int32)
    packed_ref[...] = vtxv * 2 + cond


_node_call = pl.pallas_call(
    _node_body,
    grid=(N_PAD // TCB,),
    in_specs=[pl.BlockSpec((5, TCB), lambda i: (0, i)),
              pl.BlockSpec((2, TCB), lambda i: (0, i)),
              pl.BlockSpec((2, TCB), lambda i: (0, i))],
    out_specs=[pl.BlockSpec((8, 128), lambda i: (0, 0)),
               pl.BlockSpec((1, TCB), lambda i: (0, i))],
    out_shape=[jax.ShapeDtypeStruct((8, 128), jnp.float32),
               jax.ShapeDtypeStruct((1, N_PAD), jnp.int32)],
)


def _edge_body(packed_hbm, x_hbm, eidx_hbm, gid_hbm, out_hbm,
               table, src_b0, dst_b0, x_b0, gid_b0,
               src_b1, dst_b1, x_b1, gid_b1, acc, outbuf, sems):
    cid = lax.axis_index("c")
    sid = lax.axis_index("s")
    wid = sid * NC + cid
    bufs = ((src_b0, dst_b0, x_b0, gid_b0), (src_b1, dst_b1, x_b1, gid_b1))
    pltpu.sync_copy(packed_hbm, table)

    def zacc(i, carry):
        acc[pl.ds(i * 16, 16)] = jnp.zeros((16,), jnp.float32)
        return carry
    lax.fori_loop(0, ACC_WORDS // 16, zacc, 0)
    lane = lax.iota(jnp.int32, 16)
    lane16 = lane * 16
    ones16 = jnp.full((16,), 1.0, jnp.float32)
    base_w = wid * E_PW

    def issue(c, bp):
        sb, db, xb, gb = bufs[bp]
        base = base_w + c * E_CHUNK
        pltpu.async_copy(eidx_hbm.at[pl.ds(base, E_CHUNK)], sb, sems.at[bp])
        pltpu.async_copy(eidx_hbm.at[pl.ds(E_TOTAL + base, E_CHUNK)],
                         db, sems.at[bp])
        pltpu.async_copy(x_hbm.at[pl.ds(base, E_CHUNK)], xb, sems.at[bp])
        pltpu.async_copy(gid_hbm.at[pl.ds(base, E_CHUNK)], gb, sems.at[bp])

    def wait(bp):
        sb, db, xb, gb = bufs[bp]
        pltpu.make_async_copy(eidx_hbm.at[pl.ds(0, E_CHUNK)],
                              sb, sems.at[bp]).wait()
        pltpu.make_async_copy(eidx_hbm.at[pl.ds(0, E_CHUNK)],
                              db, sems.at[bp]).wait()
        pltpu.make_async_copy(x_hbm.at[pl.ds(0, E_CHUNK)],
                              xb, sems.at[bp]).wait()
        pltpu.make_async_copy(gid_hbm.at[pl.ds(0, E_CHUNK)],
                              gb, sems.at[bp]).wait()

    def process(bp):
        sb, db, xb, gb = bufs[bp]

        @plsc.parallel_loop(0, E_STEPS, unroll=N_BANKS)
        def _(i):
            sl = pl.ds(i * 16, 16)
            s = sb[sl]
            d = db[sl]
            x = xb[sl]
            g = gb[sl]
            ps = plsc.load_gather(table, [s])
            pd = plsc.load_gather(table, [d])
            y = jnp.where((ps >> 1) == (pd >> 1), 1.0, 0.0).astype(jnp.float32)
            w = jnp.where((ps & pd & 1) == 1, 2.0, 1.0).astype(jnp.float32)
            t = jnp.exp(-jnp.abs(x))
            p = jnp.full((16,), _LOG1P_C[8], jnp.float32)
            for k in range(7, -1, -1):
                p = p * t + _LOG1P_C[k]
            r = 1.0 / (1.0 + t)
            yhat = jnp.where(x >= 0, r, 1.0 - r)
            bce = (jnp.maximum(x, 0.0) - x * y + p) * w
            yw = y * w
            yhw = yhat * w
            tp = yhat * yw
            bank = lax.rem(i, N_BANKS)
            addr = bank * (5 * 256) + lane16 + g
            plsc.addupdate_scatter(acc, [addr], bce)
            plsc.addupdate_scatter(acc, [addr + 256], ones16)
            plsc.addupdate_scatter(acc, [addr + 512], tp)
            plsc.addupdate_scatter(acc, [addr + 768], yw)
            plsc.addupdate_scatter(acc, [addr + 1024], yhw)

    issue(0, 0)
    issue(1, 1)

    def outer(cc, carry):
        c0 = 2 * cc
        wait(0)
        process(0)
        issue(jnp.minimum(c0 + 2, E_NCHUNK - 1), 0)
        wait(1)
        process(1)
        issue(jnp.minimum(c0 + 3, E_NCHUNK - 1), 1)
        return carry
    lax.fori_loop(0, E_NCHUNK // 2, outer, 0)
    wait(0)
    wait(1)

    for q in range(5):
        def red(j, v):
            off = (j // 16) * (5 * 256) + q * 256 + (j % 16) * 16
            return v + acc[pl.ds(off, 16)]
        v = lax.fori_loop(0, N_BANKS * 16, red,
                          jnp.zeros((16,), jnp.float32))
        outbuf[q, :] = v
    pltpu.sync_copy(outbuf, out_hbm.at[wid])


def _get_edge_call():
    if "edge" not in _CALL_CACHE:
        _CALL_CACHE["edge"] = pl.kernel(
            _edge_body,
            out_type=jax.ShapeDtypeStruct((NW, 5, 16), jnp.float32),
            mesh=plsc.VectorSubcoreMesh(core_axis_name="c",
                                        subcore_axis_name="s",
                                        num_cores=NC, num_subcores=NS),
            compiler_params=pltpu.CompilerParams(needs_layout_passes=False),
            scratch_types=[
                pltpu.VMEM((N_PAD,), jnp.int32),
                pltpu.VMEM((E_CHUNK,), jnp.int32),
                pltpu.VMEM((E_CHUNK,), jnp.int32),
                pltpu.VMEM((E_CHUNK,), jnp.float32),
                pltpu.VMEM((E_CHUNK,), jnp.int32),
                pltpu.VMEM((E_CHUNK,), jnp.int32),
                pltpu.VMEM((E_CHUNK,), jnp.int32),
                pltpu.VMEM((E_CHUNK,), jnp.float32),
                pltpu.VMEM((E_CHUNK,), jnp.int32),
                pltpu.VMEM((ACC_WORDS,), jnp.float32),
                pltpu.VMEM((5, 16), jnp.float32),
                pltpu.SemaphoreType.DMA((2,)),
            ],
        )
    return _CALL_CACHE["edge"]


def kernel(node_prediction, edge_prediction, edge_index, track_labels,
           track_vtx, track_node_idx, lep_labels, lep_vtx, lep_node_idx,
           cell_labels, cell_vtx, cell_node_idx, node_graph_ids,
           edge_graph_ids):
    idx = jnp.concatenate([track_node_idx, lep_node_idx, cell_node_idx])
    lab = jnp.concatenate([track_labels, lep_labels, cell_labels])
    vtx = jnp.concatenate([track_vtx, lep_vtx, cell_vtx])
    padn = ITEM_PAD - N_ITEMS
    idx = jnp.pad(idx.astype(jnp.int32), (0, padn)).reshape(IROWS, 128)
    lab = jnp.pad(lab.astype(jnp.int32), (0, padn)).reshape(IROWS, 128)
    vtx = jnp.pad(vtx.astype(jnp.int32), (0, padn)).reshape(IROWS, 128)

    lab_part, vtx_part = _get_scatter_call()(idx, lab, vtx)

    predT = jnp.pad(node_prediction, ((0, N_PAD - N_NODES), (0, 0))).T
    loss_parts, packed = _node_call(predT, lab_part, vtx_part)

    acc = _get_edge_call()(packed.reshape(N_PAD),
                     edge_prediction,
                     edge_index.astype(jnp.int32).reshape(2 * E_TOTAL),
                     edge_graph_ids.astype(jnp.int32))

    parts = acc.sum(axis=0)
    bce_sum = parts[0]
    cnt = parts[1]
    tp = parts[2]
    yw = parts[3]
    yhw = parts[4]
    fn = yw - tp
    fp = yhw - tp
    node_loss = loss_parts.sum() / N_GRAPHS
    edge_bce = (bce_sum / jnp.maximum(cnt, 1.0)).mean()
    edge_f1 = -(2.0 * tp / (2.0 * tp + fp + fn + 1e-10)).mean()
    loss = node_loss + edge_bce + edge_f1
    return (loss, node_loss, edge_bce, edge_f1)
